# Initial kernel scaffold; baseline (speedup 1.0000x reference)
import jax
import jax.numpy as jnp
from jax.experimental import pallas as pl


def kernel(x, p00, p01, p02, p03, p04, p05, p06, p07, p08, p09, p10, p11, p12, p13, p14, p15, p16, p17, p18, p19, p20, p21, p22, p23, p24, p25, p26, p27, p28, p29, p30, p31, p32, p33, p34, p35, p36, p37, p38, p39, p40, p41, p42, p43, p44, p45, p46, p47, p48, p49, p50, p51, p52, p53, p54, p55, p56, p57, p58, p59, p60, p61, p62, p63, p64, p65, p66, p67):
    raise NotImplementedError("write your pallas kernel here")


# speedup vs baseline: 1.0924x; 1.0924x over previous
import math

import jax
import jax.numpy as jnp
from jax.experimental import pallas as pl
from jax.experimental.pallas import tpu as pltpu

_VMEM_LIMIT = 56 * 1024 * 1024


def _erf(x):
    p = 0.3275911
    a1, a2, a3, a4, a5 = 0.254829592, -0.284496736, 1.421413741, -1.453152027, 1.061405429
    sign = jnp.sign(x)
    ax = jnp.abs(x)
    t = 1.0 / (1.0 + p * ax)
    poly = ((((a5 * t + a4) * t + a3) * t + a2) * t + a1) * t
    return sign * (1.0 - poly * jnp.exp(-ax * ax))


def _gelu(x):
    return 0.5 * x * (1.0 + _erf(x * (1.0 / math.sqrt(2.0))))


def _stem_kernel(a_ref, w_ref, s_ref, b_ref, o_ref):
    y = jnp.dot(a_ref[...], w_ref[...], preferred_element_type=jnp.float32)
    y = jnp.maximum(y * s_ref[...] + b_ref[...], 0.0)
    o_ref[...] = y.astype(o_ref.dtype)


def _stem_conv(x_nchw, w, scale, shift):
    xh = jnp.transpose(x_nchw, (0, 2, 3, 1)).astype(jnp.bfloat16)
    xp = jnp.pad(xh, ((0, 0), (3, 3), (3, 3), (0, 0)))
    cols = [xp[:, dy:dy + 31:2, dx:dx + 31:2, :] for dy in range(7) for dx in range(7)]
    patches = jnp.concatenate(cols, axis=-1).reshape(128 * 256, 147)
    patches = jnp.pad(patches, ((0, 0), (0, 109)))
    tm = 2048
    out = pl.pallas_call(
        _stem_kernel,
        out_shape=jax.ShapeDtypeStruct((128 * 256, 128), jnp.bfloat16),
        grid=(32768 // tm,),
        in_specs=[pl.BlockSpec((tm, 256), lambda i: (i, 0)),
                  pl.BlockSpec((256, 128), lambda i: (0, 0)),
                  pl.BlockSpec((1, 128), lambda i: (0, 0)),
                  pl.BlockSpec((1, 128), lambda i: (0, 0))],
        out_specs=pl.BlockSpec((tm, 128), lambda i: (i, 0)),
        compiler_params=pltpu.CompilerParams(
            dimension_semantics=("parallel",), vmem_limit_bytes=_VMEM_LIMIT),
    )(patches, w, scale, shift)
    return out.reshape(128, 16, 16, 128)


def _pool_kernel(ee_ref, eo_ref, oe_ref, oo_ref, o_ref):
    ee = ee_ref[...].astype(jnp.float32)
    eo = eo_ref[...].astype(jnp.float32)
    oe = oe_ref[...].astype(jnp.float32)
    oo = oo_ref[...].astype(jnp.float32)
    m = ee[:, :-1, :-1, :]
    m = jnp.maximum(m, ee[:, :-1, 1:, :])
    m = jnp.maximum(m, ee[:, 1:, :-1, :])
    m = jnp.maximum(m, ee[:, 1:, 1:, :])
    m = jnp.maximum(m, eo[:, :-1, :, :])
    m = jnp.maximum(m, eo[:, 1:, :, :])
    m = jnp.maximum(m, oe[:, :, :-1, :])
    m = jnp.maximum(m, oe[:, :, 1:, :])
    m = jnp.maximum(m, oo)
    o_ref[...] = jnp.zeros(o_ref.shape, o_ref.dtype)
    o_ref[:, 1:9, 1:9, :] = m.astype(o_ref.dtype)


def _maxpool_padded(x):
    xp = jnp.pad(x, ((0, 0), (1, 1), (1, 1), (0, 0)))
    ee = xp[:, 0::2, 0::2, :]
    eo = xp[:, 0::2, 1::2, :][:, :, :8, :]
    oe = xp[:, 1::2, 0::2, :][:, :8, :, :]
    oo = xp[:, 1::2, 1::2, :][:, :8, :8, :]
    bt = 64
    bspec = lambda h, w: pl.BlockSpec((bt, h, w, 128), lambda i: (i, 0, 0, 0))
    return pl.pallas_call(
        _pool_kernel,
        out_shape=jax.ShapeDtypeStruct((128, 10, 10, 128), x.dtype),
        grid=(128 // bt,),
        in_specs=[bspec(9, 9), bspec(9, 8), bspec(8, 9), bspec(8, 8)],
        out_specs=bspec(10, 10),
        compiler_params=pltpu.CompilerParams(
            dimension_semantics=("parallel",), vmem_limit_bytes=_VMEM_LIMIT),
    )(ee, eo, oe, oo)


def _conv3x3(src_ref, w_ref, m, hw):
    taps = [src_ref[:, dy:dy + hw, dx:dx + hw, :].reshape(m, 128)
            for dy in range(3) for dx in range(3)]
    a = jnp.concatenate(taps, axis=1)
    return jnp.dot(a, w_ref[...], preferred_element_type=jnp.float32)


def _layer1_kernel(xp_ref,
                   w11, s11, b11, w12, s12, b12,
                   w21, s21, b21, w22, s22, b22,
                   o_ref, sa, sb):
    bt = xp_ref.shape[0]
    m = bt * 64
    sa[...] = jnp.zeros(sa.shape, sa.dtype)
    sb[...] = jnp.zeros(sb.shape, sb.dtype)
    identity = xp_ref[:, 1:9, 1:9, :].reshape(m, 128).astype(jnp.float32)

    a1 = jnp.maximum(_conv3x3(xp_ref, w11, m, 8) * s11[...] + b11[...], 0.0)
    sa[:, 1:9, 1:9, :] = a1.reshape(bt, 8, 8, 128).astype(sa.dtype)
    v1 = jnp.maximum(_conv3x3(sa, w12, m, 8) * s12[...] + b12[...] + identity, 0.0)
    v1 = v1.astype(jnp.bfloat16)
    sb[:, 1:9, 1:9, :] = v1.reshape(bt, 8, 8, 128)

    a2 = jnp.maximum(_conv3x3(sb, w21, m, 8) * s21[...] + b21[...], 0.0)
    sa[:, 1:9, 1:9, :] = a2.reshape(bt, 8, 8, 128).astype(sa.dtype)
    out = jnp.maximum(_conv3x3(sa, w22, m, 8) * s22[...] + b22[...]
                      + v1.astype(jnp.float32), 0.0)
    o_ref[...] = jnp.zeros(o_ref.shape, o_ref.dtype)
    o_ref[:, 1:9, 1:9, :] = out.reshape(bt, 8, 8, 128).astype(o_ref.dtype)


def _layer1(xp, p):
    bt = 32
    wspec = pl.BlockSpec((1152, 128), lambda i: (0, 0))
    vspec = pl.BlockSpec((1, 128), lambda i: (0, 0))
    bspec = pl.BlockSpec((bt, 10, 10, 128), lambda i: (i, 0, 0, 0))
    return pl.pallas_call(
        _layer1_kernel,
        out_shape=jax.ShapeDtypeStruct((128, 10, 10, 128), jnp.bfloat16),
        grid=(128 // bt,),
        in_specs=[bspec] + [wspec, vspec, vspec] * 4,
        out_specs=bspec,
        scratch_shapes=[pltpu.VMEM((bt, 10, 10, 128), jnp.bfloat16),
                        pltpu.VMEM((bt, 10, 10, 128), jnp.bfloat16)],
        compiler_params=pltpu.CompilerParams(
            dimension_semantics=("parallel",), vmem_limit_bytes=_VMEM_LIMIT),
    )(xp, p["w11"], p["s11"], p["b11"], p["w12"], p["s12"], p["b12"],
      p["w21"], p["s21"], p["b21"], p["w22"], p["s22"], p["b22"])


def _conv3x3_4(src_ref, w_ref, m):
    taps = [src_ref[:, dy:dy + 4, dx:dx + 4, :].reshape(m, 128)
            for dy in range(3) for dx in range(3)]
    a = jnp.concatenate(taps, axis=1)
    return jnp.dot(a, w_ref[...], preferred_element_type=jnp.float32)


def _layer2_kernel(ee_ref, eo_ref, oe_ref, oo_ref,
                   wf, sf, bf, w12, s12, b12,
                   w21, s21, b21, w22, s22, b22,
                   o_ref, sa, sb):
    bt = ee_ref.shape[0]
    m = bt * 16
    planes = (ee_ref, eo_ref, oe_ref, oo_ref)
    sa[...] = jnp.zeros(sa.shape, sa.dtype)
    sb[...] = jnp.zeros(sb.shape, sb.dtype)

    taps = []
    for dy in range(3):
        for dx in range(3):
            pr = planes[(dy % 2) * 2 + (dx % 2)]
            taps.append(pr[:, dy // 2:dy // 2 + 4, dx // 2:dx // 2 + 4, :]
                        .reshape(m, 128))
    acc = jnp.dot(jnp.concatenate(taps, axis=1), wf[...],
                  preferred_element_type=jnp.float32)
    y = acc * sf[...] + bf[...]
    main = jnp.maximum(y[:, :128], 0.0)
    down = y[:, 128:].astype(jnp.bfloat16)

    sa[:, 1:5, 1:5, :] = main.reshape(bt, 4, 4, 128).astype(sa.dtype)
    v1 = jnp.maximum(_conv3x3_4(sa, w12, m) * s12[...] + b12[...]
                     + down.astype(jnp.float32), 0.0)
    v1 = v1.astype(jnp.bfloat16)
    sb[:, 1:5, 1:5, :] = v1.reshape(bt, 4, 4, 128)

    a2 = jnp.maximum(_conv3x3_4(sb, w21, m) * s21[...] + b21[...], 0.0)
    sa[:, 1:5, 1:5, :] = a2.reshape(bt, 4, 4, 128).astype(sa.dtype)
    out = jnp.maximum(_conv3x3_4(sa, w22, m) * s22[...] + b22[...]
                      + v1.astype(jnp.float32), 0.0)
    o_ref[...] = out.reshape(bt, 4, 4, 128).astype(o_ref.dtype)


def _layer2(x1p, p):
    ee = x1p[:, 0::2, 0::2, :]
    eo = x1p[:, 0::2, 1::2, :]
    oe = x1p[:, 1::2, 0::2, :]
    oo = x1p[:, 1::2, 1::2, :]
    bt = 64
    pspec = pl.BlockSpec((bt, 5, 5, 128), lambda i: (i, 0, 0, 0))
    wspec = pl.BlockSpec((1152, 128), lambda i: (0, 0))
    vspec = pl.BlockSpec((1, 128), lambda i: (0, 0))
    wfspec = pl.BlockSpec((1152, 256), lambda i: (0, 0))
    vfspec = pl.BlockSpec((1, 256), lambda i: (0, 0))
    return pl.pallas_call(
        _layer2_kernel,
        out_shape=jax.ShapeDtypeStruct((128, 4, 4, 128), jnp.bfloat16),
        grid=(128 // bt,),
        in_specs=[pspec, pspec, pspec, pspec,
                  wfspec, vfspec, vfspec,
                  wspec, vspec, vspec, wspec, vspec, vspec, wspec, vspec, vspec],
        out_specs=pl.BlockSpec((bt, 4, 4, 128), lambda i: (i, 0, 0, 0)),
        scratch_shapes=[pltpu.VMEM((bt, 6, 6, 128), jnp.bfloat16),
                        pltpu.VMEM((bt, 6, 6, 128), jnp.bfloat16)],
        compiler_params=pltpu.CompilerParams(
            dimension_semantics=("parallel",), vmem_limit_bytes=_VMEM_LIMIT),
    )(ee, eo, oe, oo, p["wf"], p["sf"], p["bf"], p["w12"], p["s12"], p["b12"],
      p["w21"], p["s21"], p["b21"], p["w22"], p["s22"], p["b22"])


def _tail_kernel(x_ref,
                 w3f, s3f, b3f, w3c, s3c, b3c,
                 w31, s31, b31, w32, s32, b32,
                 w4f, s4f, b4f, w4c, s4c, b4c,
                 w41, s41, b41, w42, s42, b42,
                 ca1, ca2, sac, vw, vb, pw, ps, pb, prs,
                 lw, ls, lb, fw, fb, o_ref):
    def mm(a, w):
        return jnp.dot(a.astype(jnp.bfloat16), w[...],
                       preferred_element_type=jnp.float32)

    xf = x_ref[...]
    y = mm(xf, w3f) * s3f[...] + b3f[...]
    main = jnp.maximum(y[:, :1024], 0.0).astype(jnp.bfloat16)
    down = y[:, 1024:].astype(jnp.bfloat16)
    x3 = jnp.maximum(mm(main, w3c) * s3c[...] + b3c[...]
                     + down.astype(jnp.float32), 0.0).astype(jnp.bfloat16)
    t = jnp.maximum(mm(x3, w31) * s31[...] + b31[...], 0.0)
    x3 = jnp.maximum(mm(t, w32) * s32[...] + b32[...]
                     + x3.astype(jnp.float32), 0.0).astype(jnp.bfloat16)
    y4 = mm(x3, w4f) * s4f[...] + b4f[...]
    m4 = jnp.maximum(y4[:, :512], 0.0).astype(jnp.bfloat16)
    d4 = y4[:, 512:].astype(jnp.bfloat16)
    x4 = jnp.maximum(mm(m4, w4c) * s4c[...] + b4c[...]
                     + d4.astype(jnp.float32), 0.0).astype(jnp.bfloat16)
    t4 = jnp.maximum(mm(x4, w41) * s41[...] + b41[...], 0.0)
    x4 = jnp.maximum(mm(t4, w42) * s42[...] + b42[...]
                     + x4.astype(jnp.float32), 0.0).astype(jnp.bfloat16)

    x = x4.astype(jnp.float32)
    h = jnp.maximum(mm(x, ca1), 0.0)
    ca = 2.0 * mm(h, ca2)
    x2 = x * jax.nn.sigmoid(ca)
    avg_c = jnp.mean(x2, axis=-1, keepdims=True)
    max_c = jnp.max(x2, axis=-1, keepdims=True)
    sa = sac[0] * avg_c + sac[1] * max_c
    x3h = x2 * jax.nn.sigmoid(sa)
    v = mm(x3h, vw) + vb[...]
    yv = mm(v, pw)
    feat = _gelu(yv * ps[...] + pb[...] + x3h * prs[...])
    y2 = mm(feat, lw)
    feat2 = _gelu(y2 * ls[...] + lb[...])
    o_ref[...] = mm(feat2, fw) + fb[...]


def _toep2x2(w):
    c = w.shape[0] // 9
    taps = [w[t * c:(t + 1) * c] for t in range(9)]
    cols = []
    for qi in range(2):
        for qj in range(2):
            rows = [taps[(pi - qi + 1) * 3 + (pj - qj + 1)]
                    for pi in range(2) for pj in range(2)]
            cols.append(jnp.concatenate(rows, axis=0))
    return jnp.concatenate(cols, axis=1)


def _toep_s2(w, cin):
    taps = [w[t * cin:(t + 1) * cin] for t in range(9)]
    zero = jnp.zeros_like(taps[0])
    cols = []
    for qi in range(2):
        for qj in range(2):
            rows = []
            for pi in range(4):
                for pj in range(4):
                    dy, dx = pi - 2 * qi + 1, pj - 2 * qj + 1
                    rows.append(taps[dy * 3 + dx] if 0 <= dy < 3 and 0 <= dx < 3
                                else zero)
            cols.append(jnp.concatenate(rows, axis=0))
    return jnp.concatenate(cols, axis=1)


def _tile4(v):
    return jnp.tile(v, (1, 4))


def _tail(xf, tp):
    bt = 64
    n_in = [pl.BlockSpec((bt, 2048), lambda i: (i, 0))]
    args = [xf]
    for a in tp["mats"]:
        if a.ndim == 1:
            n_in.append(pl.BlockSpec(memory_space=pltpu.MemorySpace.SMEM))
        else:
            r, c = a.shape
            n_in.append(pl.BlockSpec((r, c), lambda i: (0, 0)))
        args.append(a)
    out = pl.pallas_call(
        _tail_kernel,
        out_shape=jax.ShapeDtypeStruct((128, 128), jnp.float32),
        grid=(128 // bt,),
        in_specs=n_in,
        out_specs=pl.BlockSpec((bt, 128), lambda i: (i, 0)),
        compiler_params=pltpu.CompilerParams(
            dimension_semantics=("parallel",), vmem_limit_bytes=_VMEM_LIMIT),
    )(*args)
    return out[:, :7]


def kernel(x, p00, p01, p02, p03, p04, p05, p06, p07, p08, p09, p10, p11, p12, p13, p14, p15, p16, p17, p18, p19, p20, p21, p22, p23, p24, p25, p26, p27, p28, p29, p30, p31, p32, p33, p34, p35, p36, p37, p38, p39, p40, p41, p42, p43, p44, p45, p46, p47, p48, p49, p50, p51, p52, p53, p54, p55, p56, p57, p58, p59, p60, p61, p62, p63, p64, p65, p66, p67):
    xs = _stem_conv(x, p02, p00, p01)
    x1p_in = _maxpool_padded(xs)
    l1 = {"s11": p17, "b11": p18, "w11": p19, "s12": p20, "b12": p21, "w12": p22,
          "s21": p23, "b21": p24, "w21": p25, "s22": p26, "b22": p27, "w22": p28}
    x1p = _layer1(x1p_in, l1)
    l2 = {"s12": p29, "b12": p30, "w12": p31,
          "sf": p33, "bf": p34, "wf": p35,
          "s21": p36, "b21": p37, "w21": p38, "s22": p39, "b22": p40, "w22": p41}
    x2 = _layer2(x1p, l2)
    xf = x2.reshape(128, 2048)

    w3f = jnp.concatenate([_toep_s2(p48[:, :256], 128),
                           _toep_s2(p48[:, 256:], 128)], axis=1)
    s3f = jnp.concatenate([_tile4(p46[:, :256]), _tile4(p46[:, 256:])], axis=1)
    b3f = jnp.concatenate([_tile4(p47[:, :256]), _tile4(p47[:, 256:])], axis=1)
    mats = [
        w3f, s3f, b3f,
        _toep2x2(p44), _tile4(p42), _tile4(p43),
        _toep2x2(p51), _tile4(p49), _tile4(p50),
        _toep2x2(p54), _tile4(p52), _tile4(p53),
        jnp.concatenate([p61[4 * 256:6 * 256], p61[7 * 256:9 * 256]], axis=0),
        p59, p60,
        p57[4 * 512:5 * 512], p55, p56,
        p64[4 * 512:5 * 512], p62, p63,
        p67[4 * 512:5 * 512], p65, p66,
        p03, p04, p14, p16, p15, p13, p12, p10, p11,
        p09, p08, p07, p06, p05,
    ]
    return _tail(xf, {"mats": mats})
